# baseline (speedup 1.0000x reference)
import jax
import jax.numpy as jnp
from jax.experimental import pallas as pl

BATCH = 4
SEQ_LEN = 4096
EMBED_DIM = 1024
SEQ_BLOCK = 512


def _add_block(x_ref, pos_ref, o_ref):
    o_ref[...] = x_ref[...] + pos_ref[...]


def kernel(x, pos_table):
    batch, seq_len, embed_dim = x.shape
    n_seq = seq_len // SEQ_BLOCK
    pos = pos_table[:seq_len]
    return pl.pallas_call(
        _add_block,
        grid=(n_seq, batch // 2),
        in_specs=[
            pl.BlockSpec((2, SEQ_BLOCK, embed_dim), lambda i, j: (j, i, 0)),
            pl.BlockSpec((SEQ_BLOCK, embed_dim), lambda i, j: (i, 0)),
        ],
        out_specs=pl.BlockSpec((2, SEQ_BLOCK, embed_dim), lambda i, j: (j, i, 0)),
        out_shape=jax.ShapeDtypeStruct((batch, seq_len, embed_dim), x.dtype),
    )(x, pos)


# speedup vs baseline: 1.0468x; 1.0468x over previous
import jax
import jax.numpy as jnp
from jax.experimental import pallas as pl

BATCH = 4
SEQ_LEN = 4096
EMBED_DIM = 1024
SEQ_BLOCK = 1024


def _add_block(x_ref, pos_ref, o_ref):
    o_ref[...] = x_ref[...] + pos_ref[...]


def kernel(x, pos_table):
    batch, seq_len, embed_dim = x.shape
    n_seq = seq_len // SEQ_BLOCK
    pos = pos_table[:seq_len]
    return pl.pallas_call(
        _add_block,
        grid=(n_seq, batch // 2),
        in_specs=[
            pl.BlockSpec((2, SEQ_BLOCK, embed_dim), lambda i, j: (j, i, 0)),
            pl.BlockSpec((SEQ_BLOCK, embed_dim), lambda i, j: (i, 0)),
        ],
        out_specs=pl.BlockSpec((2, SEQ_BLOCK, embed_dim), lambda i, j: (j, i, 0)),
        out_shape=jax.ShapeDtypeStruct((batch, seq_len, embed_dim), x.dtype),
    )(x, pos)
